# baseline (speedup 1.0000x reference)
import functools

import jax
import jax.numpy as jnp
from jax import lax
from jax.experimental import pallas as pl
from jax.experimental.pallas import tpu as pltpu
from jax.experimental.pallas import tpu_sc as plsc

EMBED_DIM = 64

_NC = 2
_NS = 16
_NW = _NC * _NS

_CHUNK = 1024


@functools.lru_cache(maxsize=None)
def _make_gather_kernel(n_rows, d):
    assert n_rows % _NW == 0
    b_per_w = n_rows // _NW
    assert b_per_w % _CHUNK == 0
    n_chunks = b_per_w // _CHUNK
    mesh = plsc.VectorSubcoreMesh(core_axis_name="c", subcore_axis_name="s")

    @functools.partial(
        pl.kernel,
        mesh=mesh,
        compiler_params=pltpu.CompilerParams(use_tc_tiling_on_sc=False),
        out_type=jax.ShapeDtypeStruct((n_rows, d), jnp.float32),
        scratch_types=[
            pltpu.VMEM((b_per_w,), jnp.int32),
            pltpu.VMEM((_CHUNK, d), jnp.float32),
            pltpu.SemaphoreType.DMA,
        ],
    )
    def gather_kernel(table_hbm, idx_hbm, out_hbm, idx_v, rows_v, sem):
        wid = lax.axis_index("s") * _NC + lax.axis_index("c")
        base = wid * b_per_w
        pltpu.sync_copy(idx_hbm.at[pl.ds(base, b_per_w)], idx_v)

        def body(i, carry):
            off = i * _CHUNK
            pltpu.async_copy(
                table_hbm.at[idx_v.at[pl.ds(off, _CHUNK)]], rows_v, sem
            ).wait()
            pltpu.sync_copy(rows_v, out_hbm.at[pl.ds(base + off, _CHUNK)])
            return carry

        lax.fori_loop(0, n_chunks, body, 0)

    return gather_kernel


def kernel(indices, weight):
    b, t = indices.shape
    flat_idx = indices.reshape(b * t).astype(jnp.int32)
    out = _make_gather_kernel(b * t, EMBED_DIM)(weight, flat_idx)
    return out.reshape(b, t, EMBED_DIM)


# speedup vs baseline: 1.0028x; 1.0028x over previous
import functools

import jax
import jax.numpy as jnp
from jax import lax
from jax.experimental import pallas as pl
from jax.experimental.pallas import tpu as pltpu
from jax.experimental.pallas import tpu_sc as plsc

EMBED_DIM = 64

_NC = 2
_NS = 16
_NW = _NC * _NS

_CHUNK = 800


@functools.lru_cache(maxsize=None)
def _make_gather_kernel(n_rows, d):
    assert n_rows % _NW == 0
    b_per_w = n_rows // _NW
    assert b_per_w % _CHUNK == 0
    n_chunks = b_per_w // _CHUNK
    assert n_chunks % 2 == 0 and n_chunks >= 4
    n_pairs = n_chunks // 2
    mesh = plsc.VectorSubcoreMesh(core_axis_name="c", subcore_axis_name="s")

    @functools.partial(
        pl.kernel,
        mesh=mesh,
        compiler_params=pltpu.CompilerParams(use_tc_tiling_on_sc=False),
        out_type=jax.ShapeDtypeStruct((n_rows, d), jnp.float32),
        scratch_types=[
            pltpu.VMEM((b_per_w,), jnp.int32),
            pltpu.VMEM((_CHUNK, d), jnp.float32),
            pltpu.VMEM((_CHUNK, d), jnp.float32),
            pltpu.SemaphoreType.DMA,
            pltpu.SemaphoreType.DMA,
            pltpu.SemaphoreType.DMA,
            pltpu.SemaphoreType.DMA,
        ],
    )
    def gather_kernel(table_hbm, idx_hbm, out_hbm, idx_v, rows_a, rows_b,
                      gsem_a, gsem_b, wsem_a, wsem_b):
        wid = lax.axis_index("s") * _NC + lax.axis_index("c")
        base = wid * b_per_w
        pltpu.sync_copy(idx_hbm.at[pl.ds(base, b_per_w)], idx_v)

        def g_copy(i, buf, sem):
            return pltpu.make_async_copy(
                table_hbm.at[idx_v.at[pl.ds(i * _CHUNK, _CHUNK)]], buf, sem)

        def w_copy(i, buf, sem):
            return pltpu.make_async_copy(
                buf, out_hbm.at[pl.ds(base + i * _CHUNK, _CHUNK)], sem)

        g_copy(0, rows_a, gsem_a).start()

        def pair(p, carry):
            e = 2 * p
            g_copy(e + 1, rows_b, gsem_b).start()
            g_copy(e, rows_a, gsem_a).wait()
            w_copy(e, rows_a, wsem_a).start()
            g_copy(e + 1, rows_b, gsem_b).wait()
            w_copy(e + 1, rows_b, wsem_b).start()
            w_copy(e, rows_a, wsem_a).wait()
            g_copy(e + 2, rows_a, gsem_a).start()
            w_copy(e + 1, rows_b, wsem_b).wait()
            return carry

        lax.fori_loop(0, n_pairs - 1, pair, 0)

        e = n_chunks - 2
        g_copy(e + 1, rows_b, gsem_b).start()
        g_copy(e, rows_a, gsem_a).wait()
        w_copy(e, rows_a, wsem_a).start()
        g_copy(e + 1, rows_b, gsem_b).wait()
        w_copy(e + 1, rows_b, wsem_b).start()
        w_copy(e, rows_a, wsem_a).wait()
        w_copy(e + 1, rows_b, wsem_b).wait()

    return gather_kernel


def kernel(indices, weight):
    b, t = indices.shape
    flat_idx = indices.reshape(b * t).astype(jnp.int32)
    out = _make_gather_kernel(b * t, EMBED_DIM)(weight, flat_idx)
    return out.reshape(b, t, EMBED_DIM)
